# baseline (speedup 1.0000x reference)
import functools

import jax
import jax.numpy as jnp
from jax import lax
from jax.experimental import pallas as pl
from jax.experimental.pallas import tpu as pltpu
from jax.experimental.pallas import tpu_sc as plsc

N = 10000
E = 320000
D = 128

NC = 2
NS = 16
N_PAD = 10240
ROWS_PER_TILE = N_PAD // NS
CHUNK = 64
E_ROWS = 5120
E_PAD = E_ROWS * CHUNK
ROWS_PER_TILE_1SC = E_ROWS // NS
ROWS_PER_TILE_2SC = E_ROWS // (NC * NS)

IDX_BLK = 32
NBUF = 4
LAG = 2

DEG_CHUNK = 128
DEG_ROWS = E_PAD // DEG_CHUNK
DEG_RPT = DEG_ROWS // NS

ROW_BLK = 400
GRID = N // ROW_BLK


def _zero_vmem_rows(buf, nrows, ncols):
    z = jnp.zeros((16,), jnp.float32)

    def body(i, _):
        for k in range(ncols // 16):
            buf[i, pl.ds(k * 16, 16)] = z
        return 0

    lax.fori_loop(0, nrows, body, 0)


def _zero_vmem_1d(buf, n):
    z = jnp.zeros((16,), jnp.float32)

    def body(i, _):
        buf[pl.ds(i * 16, 16)] = z
        return 0

    lax.fori_loop(0, n // 16, body, 0)


def _zero_acc_stripe(rowbuf, acc, s):
    _zero_vmem_rows(rowbuf, CHUNK, D)
    for t in range(ROWS_PER_TILE // CHUNK):
        pltpu.sync_copy(
            rowbuf, acc.at[pl.ds(s * ROWS_PER_TILE + t * CHUNK, CHUNK), :])


def _edge_loop(table, edges_hbm, acc, idx_v, rowbuf,
               gsems, ssems, chunk_base, n_chunks):
    def blk_body(bi, _):
        base = chunk_base + bi * IDX_BLK
        pltpu.sync_copy(edges_hbm.at[pl.ds(base, IDX_BLK)], idx_v)

        gd = [None] * NBUF
        pend_s = [None] * NBUF
        for k in range(IDX_BLK + LAG):
            if k < IDX_BLK:
                b = k % NBUF
                if pend_s[b] is not None:
                    pend_s[b].wait()
                    pend_s[b] = None
                gd[b] = pltpu.async_copy(
                    table.at[idx_v.at[k, 0]], rowbuf.at[b], gsems[b])
            if k >= LAG:
                j = k - LAG
                bj = j % NBUF
                gd[bj].wait()
                pend_s[bj] = pltpu.async_copy(
                    rowbuf.at[bj], acc.at[idx_v.at[j, 1]], ssems[bj],
                    add=True)
        for b in range(NBUF):
            if pend_s[b] is not None:
                pend_s[b].wait()
        return 0

    lax.fori_loop(0, n_chunks // IDX_BLK, blk_body, 0)


def _copy_out_stripe(acc, out, s):
    @pl.when(s < NS - 1)
    def _():
        pltpu.sync_copy(acc.at[pl.ds(s * ROWS_PER_TILE, ROWS_PER_TILE), :],
                        out.at[pl.ds(s * ROWS_PER_TILE, ROWS_PER_TILE), :])

    @pl.when(s == NS - 1)
    def _():
        last = N - (NS - 1) * ROWS_PER_TILE
        pltpu.sync_copy(acc.at[pl.ds((NS - 1) * ROWS_PER_TILE, last), :],
                        out.at[pl.ds((NS - 1) * ROWS_PER_TILE, last), :])


def _sc_degrees(edges_deg, degs_out, degd_out, idx_v, ones_v, zero_v, acc, sem):
    c = lax.axis_index("c")
    s = lax.axis_index("s")

    one = jnp.full((16,), 1.0, jnp.float32)
    for k in range(DEG_CHUNK // 16):
        ones_v[pl.ds(k * 16, 16)] = one
    _zero_vmem_1d(zero_v, ROWS_PER_TILE)

    pltpu.sync_copy(zero_v, acc.at[pl.ds(s * ROWS_PER_TILE, ROWS_PER_TILE)])
    plsc.subcore_barrier()

    def body(j, _):
        pltpu.sync_copy(ones_v, acc.at[idx_v.at[j]], add=True)
        return 0

    @pl.when(c == 0)
    def _():
        pltpu.sync_copy(edges_deg.at[0].at[pl.ds(s * DEG_RPT, DEG_RPT)], idx_v)
        lax.fori_loop(0, DEG_RPT, body, 0)

    @pl.when(c == 1)
    def _():
        pltpu.sync_copy(edges_deg.at[1].at[pl.ds(s * DEG_RPT, DEG_RPT)], idx_v)
        lax.fori_loop(0, DEG_RPT, body, 0)

    plsc.subcore_barrier()

    stripe = pl.ds(s * ROWS_PER_TILE, ROWS_PER_TILE)

    @pl.when(c == 0)
    def _():
        pltpu.sync_copy(acc.at[stripe], degs_out.at[stripe])

    @pl.when(c == 1)
    def _():
        pltpu.sync_copy(acc.at[stripe], degd_out.at[stripe])


def _sc_segsum2(t0_hbm, t1_hbm, edges_hbm, g0_out, g1_out,
                idx_v, rowbuf, acc, *sems):
    c = lax.axis_index("c")
    s = lax.axis_index("s")
    gsems, ssems = sems[:NBUF], sems[NBUF:]

    _zero_acc_stripe(rowbuf.at[0], acc, s)
    plsc.subcore_barrier()

    @pl.when(c == 0)
    def _():
        _edge_loop(t0_hbm, edges_hbm, acc, idx_v, rowbuf,
                   gsems, ssems, s * ROWS_PER_TILE_1SC, ROWS_PER_TILE_1SC)

    @pl.when(c == 1)
    def _():
        _edge_loop(t1_hbm, edges_hbm, acc, idx_v, rowbuf,
                   gsems, ssems, s * ROWS_PER_TILE_1SC, ROWS_PER_TILE_1SC)

    plsc.subcore_barrier()

    @pl.when(c == 0)
    def _():
        _copy_out_stripe(acc, g0_out, s)

    @pl.when(c == 1)
    def _():
        _copy_out_stripe(acc, g1_out, s)


def _sc_segsum1(t_hbm, edges_hbm, g_out, idx_v, rowbuf, acc, *sems):
    c = lax.axis_index("c")
    s = lax.axis_index("s")
    w = c * NS + s
    gsems, ssems = sems[:NBUF], sems[NBUF:]

    _zero_acc_stripe(rowbuf.at[0], acc, s)
    plsc.subcore_barrier()

    _edge_loop(t_hbm, edges_hbm, acc, idx_v, rowbuf,
               gsems, ssems, w * ROWS_PER_TILE_2SC, ROWS_PER_TILE_2SC)

    plsc.subcore_barrier()
    _copy_out_stripe(acc, g_out.at[c], s)


def _tc_matmul(x_ref, hx_ref, w_ref, p_ref):
    iv = jnp.concatenate([x_ref[...], hx_ref[...]], axis=1)
    p_ref[...] = jnp.dot(iv, w_ref[...], preferred_element_type=jnp.float32)


def _tc_scalepack(p_ref, degs_ref, pr_ref, pz_ref, px_ref):
    ns = lax.rsqrt(jnp.maximum(degs_ref[...], 1.0))
    p = p_ref[...] * ns
    pr_ref[...] = p[:, :D]
    pz_ref[...] = p[:, D:2 * D]
    px_ref[...] = p[:, 2 * D:]


def _tc_rz(gr_ref, gz_ref, degd_ref, degs_ref, br_ref, bz_ref, hx_ref,
           px_ref, whh_ref, m2_ref, z_ref):
    nd = lax.rsqrt(jnp.maximum(degd_ref[...], 1.0))
    ns = lax.rsqrt(jnp.maximum(degs_ref[...], 1.0))
    r = jax.nn.sigmoid(gr_ref[...] * nd + br_ref[...])
    z = jax.nn.sigmoid(gz_ref[...] * nd + bz_ref[...])
    m2 = px_ref[...] + ns * jnp.dot(
        r * hx_ref[...], whh_ref[...], preferred_element_type=jnp.float32)
    m2_ref[...] = m2
    z_ref[...] = z


def _tc_final(gh_ref, degd_ref, bh_ref, z_ref, hx_ref, out_ref):
    nd = lax.rsqrt(jnp.maximum(degd_ref[...], 1.0))
    h = jnp.tanh((gh_ref[0] + gh_ref[1]) * nd + bh_ref[...])
    z = z_ref[...]
    out_ref[...] = z * hx_ref[...] + (1.0 - z) * h


def _row_spec(blk=ROW_BLK, cols=D):
    return pl.BlockSpec((blk, cols), lambda i: (i, 0))


def _full_spec(shape):
    nd = len(shape)
    return pl.BlockSpec(shape, lambda i: (0,) * nd)


def kernel(x, hx, edge_index, W_r, b_r, W_z, b_z, W_h, b_h):
    src = edge_index[0].astype(jnp.int32)
    dst = edge_index[1].astype(jnp.int32)

    pad = E_PAD - E
    pad_i = jnp.arange(pad, dtype=jnp.int32)
    pad_hi = N + pad_i % (N_PAD - N)
    src_p = jnp.concatenate([src, pad_i % N]).reshape(E_ROWS, 1, CHUNK)
    dst_p = jnp.concatenate([dst, pad_hi]).reshape(E_ROWS, 1, CHUNK)
    edges = jnp.concatenate([src_p, dst_p], axis=1)

    edges_deg = jnp.stack([
        jnp.concatenate([src, pad_hi]).reshape(DEG_ROWS, DEG_CHUNK),
        jnp.concatenate([dst, pad_hi]).reshape(DEG_ROWS, DEG_CHUNK),
    ])

    zeros_d = jnp.zeros((D, D), jnp.float32)
    W_cat = jnp.concatenate([
        jnp.concatenate([W_r[:D], W_z[:D], W_h[:D]], axis=1),
        jnp.concatenate([W_r[D:], W_z[D:], zeros_d], axis=1),
    ], axis=0)
    W_hh = W_h[D:]

    mesh = plsc.VectorSubcoreMesh(
        core_axis_name="c", subcore_axis_name="s", num_cores=NC, num_subcores=NS)

    P_raw = pl.pallas_call(
        _tc_matmul,
        grid=(GRID,),
        in_specs=[_row_spec(), _row_spec(), _full_spec((2 * D, 3 * D))],
        out_specs=_row_spec(cols=3 * D),
        out_shape=jax.ShapeDtypeStruct((N, 3 * D), jnp.float32),
    )(x, hx, W_cat)

    degs_pad, degd_pad = pl.kernel(
        _sc_degrees,
        out_type=(jax.ShapeDtypeStruct((N_PAD,), jnp.float32),
                  jax.ShapeDtypeStruct((N_PAD,), jnp.float32)),
        mesh=mesh,
        scratch_types=[
            pltpu.VMEM((DEG_RPT, DEG_CHUNK), jnp.int32),
            pltpu.VMEM((DEG_CHUNK,), jnp.float32),
            pltpu.VMEM((ROWS_PER_TILE,), jnp.float32),
            pltpu.VMEM_SHARED((N_PAD,), jnp.float32),
            pltpu.SemaphoreType.DMA,
        ],
    )(edges_deg)
    degs = degs_pad[:N].reshape(N, 1)
    degd = degd_pad[:N].reshape(N, 1)

    P_r, P_z, P_x = pl.pallas_call(
        _tc_scalepack,
        grid=(GRID,),
        in_specs=[_row_spec(cols=3 * D), _row_spec(cols=1)],
        out_specs=[_row_spec(), _row_spec(), _row_spec()],
        out_shape=[jax.ShapeDtypeStruct((N, D), jnp.float32),
                   jax.ShapeDtypeStruct((N, D), jnp.float32),
                   jax.ShapeDtypeStruct((N, D), jnp.float32)],
    )(P_raw, degs)

    G_r, G_z = pl.kernel(
        _sc_segsum2,
        out_type=(jax.ShapeDtypeStruct((N, D), jnp.float32),
                  jax.ShapeDtypeStruct((N, D), jnp.float32)),
        mesh=mesh,
        scratch_types=[
            pltpu.VMEM((IDX_BLK, 2, CHUNK), jnp.int32),
            pltpu.VMEM((NBUF, CHUNK, D), jnp.float32),
            pltpu.VMEM_SHARED((N_PAD, D), jnp.float32),
        ] + [pltpu.SemaphoreType.DMA] * (2 * NBUF),
    )(P_r, P_z, edges)

    M2, z_arr = pl.pallas_call(
        _tc_rz,
        grid=(GRID,),
        in_specs=[_row_spec(), _row_spec(), _row_spec(cols=1), _row_spec(cols=1),
                  _full_spec((1, D)), _full_spec((1, D)), _row_spec(),
                  _row_spec(), _full_spec((D, D))],
        out_specs=[_row_spec(), _row_spec()],
        out_shape=[jax.ShapeDtypeStruct((N, D), jnp.float32),
                   jax.ShapeDtypeStruct((N, D), jnp.float32)],
    )(G_r, G_z, degd, degs, b_r.reshape(1, D), b_z.reshape(1, D), hx, P_x, W_hh)

    G_h = pl.kernel(
        _sc_segsum1,
        out_type=jax.ShapeDtypeStruct((NC, N, D), jnp.float32),
        mesh=mesh,
        scratch_types=[
            pltpu.VMEM((IDX_BLK, 2, CHUNK), jnp.int32),
            pltpu.VMEM((NBUF, CHUNK, D), jnp.float32),
            pltpu.VMEM_SHARED((N_PAD, D), jnp.float32),
        ] + [pltpu.SemaphoreType.DMA] * (2 * NBUF),
    )(M2, edges)

    out = pl.pallas_call(
        _tc_final,
        grid=(GRID,),
        in_specs=[pl.BlockSpec((NC, ROW_BLK, D), lambda i: (0, i, 0)),
                  _row_spec(cols=1), _full_spec((1, D)), _row_spec(), _row_spec()],
        out_specs=_row_spec(),
        out_shape=jax.ShapeDtypeStruct((N, D), jnp.float32),
    )(G_h, degd, b_h.reshape(1, D), z_arr, hx)

    return out


# speedup vs baseline: 1.0299x; 1.0299x over previous
import functools

import jax
import jax.numpy as jnp
from jax import lax
from jax.experimental import pallas as pl
from jax.experimental.pallas import tpu as pltpu
from jax.experimental.pallas import tpu_sc as plsc

N = 10000
E = 320000
D = 128

NC = 2
NS = 16
N_PAD = 10240
ROWS_PER_TILE = N_PAD // NS
CHUNK = 64
E_ROWS = 5120
E_PAD = E_ROWS * CHUNK
ROWS_PER_TILE_1SC = E_ROWS // NS
ROWS_PER_TILE_2SC = E_ROWS // (NC * NS)

IDX_BLK = 32
NBUF = 4
LAG = 2

DEG_CHUNK = 128
DEG_ROWS = E_PAD // DEG_CHUNK
DEG_RPT = DEG_ROWS // NS

ROW_BLK = 400
GRID = N // ROW_BLK


def _zero_vmem_rows(buf, nrows, ncols):
    z = jnp.zeros((16,), jnp.float32)

    def body(i, _):
        for k in range(ncols // 16):
            buf[i, pl.ds(k * 16, 16)] = z
        return 0

    lax.fori_loop(0, nrows, body, 0)


def _zero_vmem_1d(buf, n):
    z = jnp.zeros((16,), jnp.float32)

    def body(i, _):
        buf[pl.ds(i * 16, 16)] = z
        return 0

    lax.fori_loop(0, n // 16, body, 0)


def _zero_acc_stripe(rowbuf, acc, s):
    _zero_vmem_rows(rowbuf, CHUNK, D)
    for t in range(ROWS_PER_TILE // CHUNK):
        pltpu.sync_copy(
            rowbuf, acc.at[pl.ds(s * ROWS_PER_TILE + t * CHUNK, CHUNK), :])


def _edge_loop(table, edges_hbm, acc, idx_v, rowbuf,
               gsems, ssems, chunk_base, n_chunks):
    def blk_body(bi, _):
        base = chunk_base + bi * IDX_BLK
        pltpu.sync_copy(edges_hbm.at[pl.ds(base, IDX_BLK)], idx_v)

        gd = [None] * NBUF
        pend_s = [None] * NBUF
        for k in range(IDX_BLK + LAG):
            if k < IDX_BLK:
                b = k % NBUF
                if pend_s[b] is not None:
                    pend_s[b].wait()
                    pend_s[b] = None
                gd[b] = pltpu.async_copy(
                    table.at[idx_v.at[k, 0]], rowbuf.at[b], gsems[b])
            if k >= LAG:
                j = k - LAG
                bj = j % NBUF
                gd[bj].wait()
                pend_s[bj] = pltpu.async_copy(
                    rowbuf.at[bj], acc.at[idx_v.at[j, 1]], ssems[bj],
                    add=True)
        for b in range(NBUF):
            if pend_s[b] is not None:
                pend_s[b].wait()
        return 0

    lax.fori_loop(0, n_chunks // IDX_BLK, blk_body, 0)


def _copy_out_stripe(acc, out, s):
    @pl.when(s < NS - 1)
    def _():
        pltpu.sync_copy(acc.at[pl.ds(s * ROWS_PER_TILE, ROWS_PER_TILE), :],
                        out.at[pl.ds(s * ROWS_PER_TILE, ROWS_PER_TILE), :])

    @pl.when(s == NS - 1)
    def _():
        last = N - (NS - 1) * ROWS_PER_TILE
        pltpu.sync_copy(acc.at[pl.ds((NS - 1) * ROWS_PER_TILE, last), :],
                        out.at[pl.ds((NS - 1) * ROWS_PER_TILE, last), :])


def _sc_degrees(edges_deg, degs_out, degd_out, idx_v, ones_v, zero_v, acc, sem):
    c = lax.axis_index("c")
    s = lax.axis_index("s")

    one = jnp.full((16,), 1.0, jnp.float32)
    for k in range(DEG_CHUNK // 16):
        ones_v[pl.ds(k * 16, 16)] = one
    _zero_vmem_1d(zero_v, ROWS_PER_TILE)

    pltpu.sync_copy(zero_v, acc.at[pl.ds(s * ROWS_PER_TILE, ROWS_PER_TILE)])
    plsc.subcore_barrier()

    def body(j, _):
        pltpu.sync_copy(ones_v, acc.at[idx_v.at[j]], add=True)
        return 0

    @pl.when(c == 0)
    def _():
        pltpu.sync_copy(edges_deg.at[0].at[pl.ds(s * DEG_RPT, DEG_RPT)], idx_v)
        lax.fori_loop(0, DEG_RPT, body, 0)

    @pl.when(c == 1)
    def _():
        pltpu.sync_copy(edges_deg.at[1].at[pl.ds(s * DEG_RPT, DEG_RPT)], idx_v)
        lax.fori_loop(0, DEG_RPT, body, 0)

    plsc.subcore_barrier()

    stripe = pl.ds(s * ROWS_PER_TILE, ROWS_PER_TILE)

    @pl.when(c == 0)
    def _():
        pltpu.sync_copy(acc.at[stripe], degs_out.at[stripe])

    @pl.when(c == 1)
    def _():
        pltpu.sync_copy(acc.at[stripe], degd_out.at[stripe])


def _sc_segsum2(t0_hbm, t1_hbm, edges_hbm, g0_out, g1_out,
                idx_v, rowbuf, acc, *sems):
    c = lax.axis_index("c")
    s = lax.axis_index("s")
    gsems, ssems = sems[:NBUF], sems[NBUF:]

    _zero_acc_stripe(rowbuf.at[0], acc, s)
    plsc.subcore_barrier()

    @pl.when(c == 0)
    def _():
        _edge_loop(t0_hbm, edges_hbm, acc, idx_v, rowbuf,
                   gsems, ssems, s * ROWS_PER_TILE_1SC, ROWS_PER_TILE_1SC)

    @pl.when(c == 1)
    def _():
        _edge_loop(t1_hbm, edges_hbm, acc, idx_v, rowbuf,
                   gsems, ssems, s * ROWS_PER_TILE_1SC, ROWS_PER_TILE_1SC)

    plsc.subcore_barrier()

    @pl.when(c == 0)
    def _():
        _copy_out_stripe(acc, g0_out, s)

    @pl.when(c == 1)
    def _():
        _copy_out_stripe(acc, g1_out, s)


def _sc_segsum1(t_hbm, edges_hbm, g_out, idx_v, rowbuf, acc, *sems):
    c = lax.axis_index("c")
    s = lax.axis_index("s")
    w = c * NS + s
    gsems, ssems = sems[:NBUF], sems[NBUF:]

    _zero_acc_stripe(rowbuf.at[0], acc, s)
    plsc.subcore_barrier()

    _edge_loop(t_hbm, edges_hbm, acc, idx_v, rowbuf,
               gsems, ssems, w * ROWS_PER_TILE_2SC, ROWS_PER_TILE_2SC)

    plsc.subcore_barrier()
    _copy_out_stripe(acc, g_out.at[c], s)


def _tc_tables(x_ref, hx_ref, w_ref, degs_ref, pr_ref, pz_ref, px_ref):
    iv = jnp.concatenate([x_ref[...], hx_ref[...]], axis=1)
    p = jnp.dot(iv, w_ref[...], preferred_element_type=jnp.float32)
    ns = lax.rsqrt(jnp.maximum(degs_ref[...], 1.0))
    p = p * ns
    pr_ref[...] = p[:, :D]
    pz_ref[...] = p[:, D:2 * D]
    px_ref[...] = p[:, 2 * D:]


def _tc_rz(gr_ref, gz_ref, degd_ref, degs_ref, br_ref, bz_ref, hx_ref,
           px_ref, whh_ref, m2_ref, z_ref):
    nd = lax.rsqrt(jnp.maximum(degd_ref[...], 1.0))
    ns = lax.rsqrt(jnp.maximum(degs_ref[...], 1.0))
    r = jax.nn.sigmoid(gr_ref[...] * nd + br_ref[...])
    z = jax.nn.sigmoid(gz_ref[...] * nd + bz_ref[...])
    m2 = px_ref[...] + ns * jnp.dot(
        r * hx_ref[...], whh_ref[...], preferred_element_type=jnp.float32)
    m2_ref[...] = m2
    z_ref[...] = z


def _tc_final(gh_ref, degd_ref, bh_ref, z_ref, hx_ref, out_ref):
    nd = lax.rsqrt(jnp.maximum(degd_ref[...], 1.0))
    h = jnp.tanh((gh_ref[0] + gh_ref[1]) * nd + bh_ref[...])
    z = z_ref[...]
    out_ref[...] = z * hx_ref[...] + (1.0 - z) * h


def _row_spec(blk=ROW_BLK, cols=D):
    return pl.BlockSpec((blk, cols), lambda i: (i, 0))


def _full_spec(shape):
    nd = len(shape)
    return pl.BlockSpec(shape, lambda i: (0,) * nd)


def kernel(x, hx, edge_index, W_r, b_r, W_z, b_z, W_h, b_h):
    src = edge_index[0].astype(jnp.int32)
    dst = edge_index[1].astype(jnp.int32)

    pad = E_PAD - E
    pad_i = jnp.arange(pad, dtype=jnp.int32)
    pad_hi = N + pad_i % (N_PAD - N)
    src_p = jnp.concatenate([src, pad_i % N]).reshape(E_ROWS, 1, CHUNK)
    dst_p = jnp.concatenate([dst, pad_hi]).reshape(E_ROWS, 1, CHUNK)
    edges = jnp.concatenate([src_p, dst_p], axis=1)

    edges_deg = jnp.stack([
        jnp.concatenate([src, pad_hi]).reshape(DEG_ROWS, DEG_CHUNK),
        jnp.concatenate([dst, pad_hi]).reshape(DEG_ROWS, DEG_CHUNK),
    ])

    zeros_d = jnp.zeros((D, D), jnp.float32)
    W_cat = jnp.concatenate([
        jnp.concatenate([W_r[:D], W_z[:D], W_h[:D]], axis=1),
        jnp.concatenate([W_r[D:], W_z[D:], zeros_d], axis=1),
    ], axis=0)
    W_hh = W_h[D:]

    mesh = plsc.VectorSubcoreMesh(
        core_axis_name="c", subcore_axis_name="s", num_cores=NC, num_subcores=NS)

    degs_pad, degd_pad = pl.kernel(
        _sc_degrees,
        out_type=(jax.ShapeDtypeStruct((N_PAD,), jnp.float32),
                  jax.ShapeDtypeStruct((N_PAD,), jnp.float32)),
        mesh=mesh,
        scratch_types=[
            pltpu.VMEM((DEG_RPT, DEG_CHUNK), jnp.int32),
            pltpu.VMEM((DEG_CHUNK,), jnp.float32),
            pltpu.VMEM((ROWS_PER_TILE,), jnp.float32),
            pltpu.VMEM_SHARED((N_PAD,), jnp.float32),
            pltpu.SemaphoreType.DMA,
        ],
    )(edges_deg)
    degs = degs_pad[:N].reshape(N, 1)
    degd = degd_pad[:N].reshape(N, 1)

    P_r, P_z, P_x = pl.pallas_call(
        _tc_tables,
        grid=(GRID,),
        in_specs=[_row_spec(), _row_spec(), _full_spec((2 * D, 3 * D)),
                  _row_spec(cols=1)],
        out_specs=[_row_spec(), _row_spec(), _row_spec()],
        out_shape=[jax.ShapeDtypeStruct((N, D), jnp.float32),
                   jax.ShapeDtypeStruct((N, D), jnp.float32),
                   jax.ShapeDtypeStruct((N, D), jnp.float32)],
    )(x, hx, W_cat, degs)

    G_r, G_z = pl.kernel(
        _sc_segsum2,
        out_type=(jax.ShapeDtypeStruct((N, D), jnp.float32),
                  jax.ShapeDtypeStruct((N, D), jnp.float32)),
        mesh=mesh,
        scratch_types=[
            pltpu.VMEM((IDX_BLK, 2, CHUNK), jnp.int32),
            pltpu.VMEM((NBUF, CHUNK, D), jnp.float32),
            pltpu.VMEM_SHARED((N_PAD, D), jnp.float32),
        ] + [pltpu.SemaphoreType.DMA] * (2 * NBUF),
    )(P_r, P_z, edges)

    M2, z_arr = pl.pallas_call(
        _tc_rz,
        grid=(GRID,),
        in_specs=[_row_spec(), _row_spec(), _row_spec(cols=1), _row_spec(cols=1),
                  _full_spec((1, D)), _full_spec((1, D)), _row_spec(),
                  _row_spec(), _full_spec((D, D))],
        out_specs=[_row_spec(), _row_spec()],
        out_shape=[jax.ShapeDtypeStruct((N, D), jnp.float32),
                   jax.ShapeDtypeStruct((N, D), jnp.float32)],
    )(G_r, G_z, degd, degs, b_r.reshape(1, D), b_z.reshape(1, D), hx, P_x, W_hh)

    G_h = pl.kernel(
        _sc_segsum1,
        out_type=jax.ShapeDtypeStruct((NC, N, D), jnp.float32),
        mesh=mesh,
        scratch_types=[
            pltpu.VMEM((IDX_BLK, 2, CHUNK), jnp.int32),
            pltpu.VMEM((NBUF, CHUNK, D), jnp.float32),
            pltpu.VMEM_SHARED((N_PAD, D), jnp.float32),
        ] + [pltpu.SemaphoreType.DMA] * (2 * NBUF),
    )(M2, edges)

    out = pl.pallas_call(
        _tc_final,
        grid=(GRID,),
        in_specs=[pl.BlockSpec((NC, ROW_BLK, D), lambda i: (0, i, 0)),
                  _row_spec(cols=1), _full_spec((1, D)), _row_spec(), _row_spec()],
        out_specs=_row_spec(),
        out_shape=jax.ShapeDtypeStruct((N, D), jnp.float32),
    )(G_h, degd, b_h.reshape(1, D), z_arr, hx)

    return out
